# baseline (speedup 1.0000x reference)
import functools

import jax
import jax.numpy as jnp
from jax.experimental import pallas as pl
from jax.experimental.pallas import tpu as pltpu


@functools.lru_cache(maxsize=None)
def _build_tc_kernel(B, H, W, C, HB):
    def body(colB_ref, rowB_ref, out_ref):
        out_ref[:, :, :, 0:C] = jnp.broadcast_to(colB_ref[...], (HB, W, B, C))
        out_ref[:, :, :, C : 2 * C] = jnp.broadcast_to(
            rowB_ref[...], (HB, W, B, C)
        )

    return pl.pallas_call(
        body,
        grid=(H // HB,),
        in_specs=[
            pl.BlockSpec((1, W, 1, C), lambda h: (0, 0, 0, 0)),
            pl.BlockSpec((HB, 1, 1, C), lambda h: (h, 0, 0, 0)),
        ],
        out_specs=pl.BlockSpec((HB, W, B, 2 * C), lambda h: (h, 0, 0, 0)),
        out_shape=jax.ShapeDtypeStruct((H, W, B, 2 * C), jnp.float32),
        compiler_params=pltpu.CompilerParams(
            dimension_semantics=("arbitrary",),
        ),
    )


def kernel(mask, row_embed, col_embed):
    B, H, W = mask.shape
    C = col_embed.shape[1]
    colB = col_embed.reshape(1, W, 1, C)
    rowB = row_embed.reshape(H, 1, 1, C)
    out_hwbc = _build_tc_kernel(B, H, W, C, 10)(colB, rowB)
    return jnp.transpose(out_hwbc, (2, 3, 0, 1))


# speedup vs baseline: 1.0673x; 1.0673x over previous
import functools

import jax
import jax.numpy as jnp
from jax.experimental import pallas as pl
from jax.experimental.pallas import tpu as pltpu


@functools.lru_cache(maxsize=None)
def _build_tc_kernel(B, H, W, C, HB):
    def body(colB_ref, rowB_ref, out_ref):
        out_ref[:, :, :, 0:C] = jnp.broadcast_to(colB_ref[...], (HB, W, B, C))
        out_ref[:, :, :, C : 2 * C] = jnp.broadcast_to(
            rowB_ref[...], (HB, W, B, C)
        )

    return pl.pallas_call(
        body,
        grid=(H // HB,),
        in_specs=[
            pl.BlockSpec((1, W, 1, C), lambda h: (0, 0, 0, 0)),
            pl.BlockSpec((HB, 1, 1, C), lambda h: (h, 0, 0, 0)),
        ],
        out_specs=pl.BlockSpec((HB, W, B, 2 * C), lambda h: (h, 0, 0, 0)),
        out_shape=jax.ShapeDtypeStruct((H, W, B, 2 * C), jnp.float32),
        compiler_params=pltpu.CompilerParams(
            dimension_semantics=("parallel",),
        ),
    )


def kernel(mask, row_embed, col_embed):
    B, H, W = mask.shape
    C = col_embed.shape[1]
    colB = col_embed.reshape(1, W, 1, C)
    rowB = row_embed.reshape(H, 1, 1, C)
    out_hwbc = _build_tc_kernel(B, H, W, C, 5)(colB, rowB)
    return jnp.transpose(out_hwbc, (2, 3, 0, 1))
